# baseline (speedup 1.0000x reference)
import functools

import jax
import jax.numpy as jnp
from jax.experimental import pallas as pl
from jax.experimental.pallas import tpu as pltpu

_SEQ = 8192
_K = 1000
_KP = 1024
_BLK = 256
_EPS = 1e-30

_ROT_A = (13, 15, 26, 6)
_ROT_B = (17, 29, 16, 24)


def _threefry_gumbel(cnt):
    ks0 = jnp.uint32(0)
    ks1 = jnp.uint32(42)
    ks2 = jnp.uint32(0x1BD11BDA) ^ ks0 ^ ks1
    ks = (ks0, ks1, ks2)
    x0 = jnp.full(cnt.shape, ks0, dtype=jnp.uint32)
    x1 = cnt + ks1

    def rotl(v, r):
        return (v << jnp.uint32(r)) | (v >> jnp.uint32(32 - r))

    for g in range(5):
        for r in (_ROT_A if g % 2 == 0 else _ROT_B):
            x0 = x0 + x1
            x1 = rotl(x1, r) ^ x0
        x0 = x0 + ks[(g + 1) % 3]
        x1 = x1 + ks[(g + 2) % 3] + jnp.uint32(g + 1)
    bits = x0 ^ x1

    tiny = jnp.float32(jnp.finfo(jnp.float32).tiny)
    u = jax.lax.bitcast_convert_type(
        (bits >> jnp.uint32(9)) | jnp.uint32(0x3F800000), jnp.float32
    ) - jnp.float32(1.0)
    u = jnp.maximum(tiny, u * (jnp.float32(1.0) - tiny) + tiny)
    return -jnp.log(-jnp.log(u))


def _body(x_ref, qt_ref, out_ref, hi_ref, mid_ref, lo_ref):
    b = pl.program_id(0)

    @pl.when(b == 0)
    def _init():
        logp = jnp.log(qt_ref[...] + jnp.float32(_EPS))
        hi = logp.astype(jnp.bfloat16)
        r1 = logp - hi.astype(jnp.float32)
        mid = r1.astype(jnp.bfloat16)
        lo = (r1 - mid.astype(jnp.float32)).astype(jnp.bfloat16)
        hi_ref[...] = hi
        mid_ref[...] = mid
        lo_ref[...] = lo

    ids = x_ref[...]
    cols = jax.lax.broadcasted_iota(jnp.int32, (_BLK, _KP), 1)
    onehot = (cols == ids).astype(jnp.bfloat16)
    dh = jnp.dot(onehot, hi_ref[...], preferred_element_type=jnp.float32)
    dm = jnp.dot(onehot, mid_ref[...], preferred_element_type=jnp.float32)
    dl = jnp.dot(onehot, lo_ref[...], preferred_element_type=jnp.float32)
    rows = dh + (dm + dl)

    rows_iota = jax.lax.broadcasted_iota(jnp.int32, (_BLK, _KP), 0)
    cnt = ((b * _BLK + rows_iota) * _K + cols).astype(jnp.uint32)
    logits = rows + _threefry_gumbel(cnt)
    logits = jnp.where(cols < _K, logits, jnp.float32(-3e38))

    m = jnp.max(logits, axis=1, keepdims=True)
    idx = jnp.min(jnp.where(logits == m, cols, _KP), axis=1, keepdims=True)
    out_ref[...] = idx.astype(jnp.int32)


@functools.partial(jax.jit, static_argnames=())
def kernel(x, t, qtcum):
    qt = qtcum[t]
    qt = jnp.pad(qt, ((0, _KP - _K), (0, _KP - _K)))
    ids = x.reshape(_SEQ, 1).astype(jnp.int32)

    out = pl.pallas_call(
        _body,
        grid=(_SEQ // _BLK,),
        in_specs=[
            pl.BlockSpec((_BLK, 1), lambda b: (b, 0)),
            pl.BlockSpec((_KP, _KP), lambda b: (0, 0)),
        ],
        out_specs=pl.BlockSpec((_BLK, 1), lambda b: (b, 0)),
        out_shape=jax.ShapeDtypeStruct((_SEQ, 1), jnp.int32),
        scratch_shapes=[
            pltpu.VMEM((_KP, _KP), jnp.bfloat16),
            pltpu.VMEM((_KP, _KP), jnp.bfloat16),
            pltpu.VMEM((_KP, _KP), jnp.bfloat16),
        ],
    )(ids, qt)
    return out.reshape(1, _SEQ)


# speedup vs baseline: 1.0254x; 1.0254x over previous
import functools

import jax
import jax.numpy as jnp
from jax.experimental import pallas as pl
from jax.experimental.pallas import tpu as pltpu

_SEQ = 8192
_K = 1000
_KP = 1024
_BLK = 512
_EPS = 1e-30

_ROT_A = (13, 15, 26, 6)
_ROT_B = (17, 29, 16, 24)


def _threefry_gumbel(cnt):
    ks0 = jnp.uint32(0)
    ks1 = jnp.uint32(42)
    ks2 = jnp.uint32(0x1BD11BDA) ^ ks0 ^ ks1
    ks = (ks0, ks1, ks2)
    x0 = jnp.full(cnt.shape, ks0, dtype=jnp.uint32)
    x1 = cnt + ks1

    def rotl(v, r):
        return (v << jnp.uint32(r)) | (v >> jnp.uint32(32 - r))

    for g in range(5):
        for r in (_ROT_A if g % 2 == 0 else _ROT_B):
            x0 = x0 + x1
            x1 = rotl(x1, r) ^ x0
        x0 = x0 + ks[(g + 1) % 3]
        x1 = x1 + ks[(g + 2) % 3] + jnp.uint32(g + 1)
    bits = x0 ^ x1

    tiny = jnp.float32(jnp.finfo(jnp.float32).tiny)
    u = jax.lax.bitcast_convert_type(
        (bits >> jnp.uint32(9)) | jnp.uint32(0x3F800000), jnp.float32
    ) - jnp.float32(1.0)
    u = jnp.maximum(tiny, u * (jnp.float32(1.0) - tiny) + tiny)
    return -jnp.log(-jnp.log(u))


def _body(x_ref, qt_ref, out_ref, hi_ref, mid_ref, lo_ref):
    b = pl.program_id(0)

    @pl.when(b == 0)
    def _init():
        logp = jnp.log(qt_ref[...] + jnp.float32(_EPS))
        hi = logp.astype(jnp.bfloat16)
        r1 = logp - hi.astype(jnp.float32)
        mid = r1.astype(jnp.bfloat16)
        lo = (r1 - mid.astype(jnp.float32)).astype(jnp.bfloat16)
        hi_ref[...] = hi
        mid_ref[...] = mid
        lo_ref[...] = lo

    ids = x_ref[...]
    cols = jax.lax.broadcasted_iota(jnp.int32, (_BLK, _KP), 1)
    onehot = (cols == ids).astype(jnp.bfloat16)
    dh = jnp.dot(onehot, hi_ref[...], preferred_element_type=jnp.float32)
    dm = jnp.dot(onehot, mid_ref[...], preferred_element_type=jnp.float32)
    dl = jnp.dot(onehot, lo_ref[...], preferred_element_type=jnp.float32)
    rows = dh + (dm + dl)

    rows_iota = jax.lax.broadcasted_iota(jnp.int32, (_BLK, _KP), 0)
    cnt = ((b * _BLK + rows_iota) * _K + cols).astype(jnp.uint32)
    logits = rows + _threefry_gumbel(cnt)

    m = jnp.max(logits, axis=1, keepdims=True)
    idx = jnp.min(jnp.where(logits == m, cols, _KP), axis=1, keepdims=True)
    out_ref[...] = idx.astype(jnp.int32)


@functools.partial(jax.jit, static_argnames=())
def kernel(x, t, qtcum):
    qt = qtcum[t]
    qt = jnp.pad(qt, ((0, _KP - _K), (0, _KP - _K)))
    ids = x.reshape(_SEQ, 1).astype(jnp.int32)

    out = pl.pallas_call(
        _body,
        grid=(_SEQ // _BLK,),
        in_specs=[
            pl.BlockSpec((_BLK, 1), lambda b: (b, 0)),
            pl.BlockSpec((_KP, _KP), lambda b: (0, 0)),
        ],
        out_specs=pl.BlockSpec((_BLK, 1), lambda b: (b, 0)),
        out_shape=jax.ShapeDtypeStruct((_SEQ, 1), jnp.int32),
        scratch_shapes=[
            pltpu.VMEM((_KP, _KP), jnp.bfloat16),
            pltpu.VMEM((_KP, _KP), jnp.bfloat16),
            pltpu.VMEM((_KP, _KP), jnp.bfloat16),
        ],
    )(ids, qt)
    return out.reshape(1, _SEQ)
